# baseline (speedup 1.0000x reference)
import jax
import jax.numpy as jnp
from jax.experimental import pallas as pl
from jax.experimental.pallas import tpu as pltpu

NUM_BAGS = 16
BM = 4096
NB = 8


def _kernel(ids_ref, w1_ref, b1_ref, w2_ref, b2_ref, x_hbm, out_hbm,
            xbuf, obuf, sums_ref, counts_ref, insem, outsem):
    b2row = b2_ref[...]
    obuf[...] = jnp.broadcast_to(b2row, obuf.shape)
    out_copies = [
        pltpu.make_async_copy(obuf, out_hbm.at[pl.ds(j * BM, BM)], outsem)
        for j in range(1, NB)
    ]
    for c in out_copies:
        c.start()

    def in_copy(c):
        return pltpu.make_async_copy(
            x_hbm.at[pl.ds(c * BM, BM)], xbuf.at[c % 2], insem.at[c % 2])

    in_copy(0).start()
    for c in range(NB):
        if c + 1 < NB:
            in_copy(c + 1).start()
        in_copy(c).wait()
        h = jnp.dot(xbuf[c % 2], w1_ref[...],
                    preferred_element_type=jnp.float32)
        h = jnp.maximum(h + b1_ref[...], 0.0)
        ids = ids_ref[c]
        onehot = (jax.lax.broadcasted_iota(jnp.int32, (NUM_BAGS, BM), 0)
                  == ids).astype(jnp.float32)
        part = jnp.dot(onehot, h, preferred_element_type=jnp.float32)
        cnt = jnp.broadcast_to(jnp.sum(onehot, axis=1, keepdims=True),
                               counts_ref.shape)
        if c == 0:
            sums_ref[...] = part
            counts_ref[...] = cnt
        else:
            sums_ref[...] += part
            counts_ref[...] += cnt

    means = sums_ref[...] / jnp.maximum(counts_ref[:, 0:1], 1.0)
    top = jnp.dot(means, w2_ref[...], preferred_element_type=jnp.float32)
    for c in out_copies:
        c.wait()
    obuf[0:NUM_BAGS, :] = top + b2row
    last = pltpu.make_async_copy(obuf, out_hbm.at[pl.ds(0, BM)], outsem)
    last.start()
    last.wait()


def kernel(x, ids, W1, b1, W2, b2):
    n, d = x.shape
    d_out = W2.shape[1]
    ids3 = ids.reshape(NB, 1, BM)
    b1r = b1.reshape(1, d)
    b2r = b2.reshape(1, d_out)

    out = pl.pallas_call(
        _kernel,
        grid=(1,),
        in_specs=[
            pl.BlockSpec((NB, 1, BM), lambda i: (0, 0, 0)),
            pl.BlockSpec((d, d), lambda i: (0, 0)),
            pl.BlockSpec((1, d), lambda i: (0, 0)),
            pl.BlockSpec((d, d_out), lambda i: (0, 0)),
            pl.BlockSpec((1, d_out), lambda i: (0, 0)),
            pl.BlockSpec(memory_space=pl.ANY),
        ],
        out_specs=pl.BlockSpec(memory_space=pl.ANY),
        out_shape=jax.ShapeDtypeStruct((n, d_out), jnp.float32),
        scratch_shapes=[
            pltpu.VMEM((2, BM, d), jnp.float32),
            pltpu.VMEM((BM, d_out), jnp.float32),
            pltpu.VMEM((NUM_BAGS, d), jnp.float32),
            pltpu.VMEM((NUM_BAGS, 128), jnp.float32),
            pltpu.SemaphoreType.DMA((2,)),
            pltpu.SemaphoreType.DMA,
        ],
    )(ids3, W1, b1r, W2, b2r, x)
    return out


# speedup vs baseline: 1.0051x; 1.0051x over previous
import jax
import jax.numpy as jnp
from jax.experimental import pallas as pl
from jax.experimental.pallas import tpu as pltpu

NUM_BAGS = 16
BM = 4096


def _fused_kernel(ids_ref, x_ref, w1_ref, b1_ref, w2_ref, b2_ref,
                  out_ref, sums_ref, counts_ref):
    i = pl.program_id(0)
    nb = pl.num_programs(0)
    h = jnp.dot(x_ref[...], w1_ref[...],
                preferred_element_type=jnp.float32)
    h = jnp.maximum(h + b1_ref[...], 0.0)
    ids = ids_ref[0]
    onehot = (jax.lax.broadcasted_iota(jnp.int32, (NUM_BAGS, BM), 0)
              == ids).astype(jnp.float32)
    part = jnp.dot(onehot, h, preferred_element_type=jnp.float32)
    cnt = jnp.broadcast_to(jnp.sum(onehot, axis=1, keepdims=True),
                           counts_ref.shape)

    @pl.when(i == 0)
    def _init():
        sums_ref[...] = part
        counts_ref[...] = cnt

    @pl.when(i != 0)
    def _acc():
        sums_ref[...] += part
        counts_ref[...] += cnt

    out_ref[...] = jnp.broadcast_to(b2_ref[...], out_ref.shape)

    @pl.when(i == nb - 1)
    def _top():
        means = sums_ref[...] / jnp.maximum(counts_ref[:, 0:1], 1.0)
        top = jnp.dot(means, w2_ref[...], preferred_element_type=jnp.float32)
        out_ref[0:NUM_BAGS, :] = top + b2_ref[...]


def kernel(x, ids, W1, b1, W2, b2):
    n, d = x.shape
    d_out = W2.shape[1]
    nb = n // BM
    ids3 = ids.reshape(nb, 1, BM)
    b1r = b1.reshape(1, d)
    b2r = b2.reshape(1, d_out)

    out = pl.pallas_call(
        _fused_kernel,
        grid=(nb,),
        in_specs=[
            pl.BlockSpec((1, 1, BM), lambda i: (i, 0, 0)),
            pl.BlockSpec((BM, d), lambda i: (i, 0)),
            pl.BlockSpec((d, d), lambda i: (0, 0)),
            pl.BlockSpec((1, d), lambda i: (0, 0)),
            pl.BlockSpec((d, d_out), lambda i: (0, 0)),
            pl.BlockSpec((1, d_out), lambda i: (0, 0)),
        ],
        out_specs=pl.BlockSpec((BM, d_out), lambda i: (pl.num_programs(0) - 1 - i, 0)),
        out_shape=jax.ShapeDtypeStruct((n, d_out), jnp.float32),
        scratch_shapes=[
            pltpu.VMEM((NUM_BAGS, d), jnp.float32),
            pltpu.VMEM((NUM_BAGS, 128), jnp.float32),
        ],
    )(ids3, x, W1, b1r, W2, b2r)
    return out
